# Initial kernel scaffold; baseline (speedup 1.0000x reference)
import jax
import jax.numpy as jnp
from jax.experimental import pallas as pl


def kernel(x, tf):
    raise NotImplementedError("write your pallas kernel here")


# speedup vs baseline: 6334.6647x; 6334.6647x over previous
import functools

import jax
import jax.numpy as jnp
from jax import lax
from jax.experimental import pallas as pl
from jax.experimental.pallas import tpu as pltpu, tpu_sc as plsc

_LANES = 16


def _sc_tf_apply(x_flat, tf_flat, *, nb, nc, res, vox):
    n_workers = 32
    per_w = (nb * vox) // n_workers
    workers_per_batch = n_workers // nb
    ch = 8192
    n_chunks = per_w // ch
    scale = float(res - 1)

    mesh = plsc.VectorSubcoreMesh(core_axis_name="c", subcore_axis_name="s")

    @functools.partial(
        pl.kernel,
        mesh=mesh,
        out_type=jax.ShapeDtypeStruct((nb * nc * vox,), jnp.float32),
        scratch_types=[
            pltpu.VMEM((nc * res,), jnp.float32),
            pltpu.VMEM((ch,), jnp.float32),
            pltpu.VMEM((nc, ch), jnp.float32),
        ],
        compiler_params=pltpu.CompilerParams(needs_layout_passes=False),
    )
    def body(x_hbm, tf_hbm, out_hbm, tfv, xbuf, obuf):
        wid = lax.axis_index("s") * 2 + lax.axis_index("c")
        n = wid // workers_per_batch
        k = wid % workers_per_batch
        x_off = n * vox + k * (vox // workers_per_batch)

        pltpu.sync_copy(tf_hbm.at[pl.ds(n * (nc * res), nc * res)], tfv)

        def chunk_body(g, _):
            src = x_off + g * ch
            pltpu.sync_copy(x_hbm.at[pl.ds(src, ch)], xbuf)

            def vec_body(i, _):
                xv = xbuf[pl.ds(i * _LANES, _LANES)]
                t = xv * scale
                idx = jnp.clip(t.astype(jnp.int32), 0, res - 2)
                frac = t - idx.astype(jnp.float32)
                for c in range(nc):
                    base = idx + (c * res)
                    y0 = plsc.load_gather(tfv, [base])
                    y1 = plsc.load_gather(tfv, [base + 1])
                    obuf[c, pl.ds(i * _LANES, _LANES)] = y0 + (y1 - y0) * frac
                return 0

            lax.fori_loop(0, ch // _LANES, vec_body, 0)

            for c in range(nc):
                dst = (n * nc + c) * vox + k * (vox // workers_per_batch) + g * ch
                pltpu.sync_copy(obuf.at[c], out_hbm.at[pl.ds(dst, ch)])
            return 0

        lax.fori_loop(0, n_chunks, chunk_body, 0)

    return body(x_flat, tf_flat)


def kernel(x, tf):
    nb, nc, res = tf.shape
    vox = x.shape[-3] * x.shape[-2] * x.shape[-1]
    out_flat = _sc_tf_apply(
        x.reshape(-1).astype(jnp.float32),
        tf.reshape(-1).astype(jnp.float32),
        nb=nb, nc=nc, res=res, vox=vox,
    )
    out_shape = (nb, nc) + x.shape[-3:]
    return out_flat.reshape(out_shape).astype(x.dtype)
